# baseline (speedup 1.0000x reference)
import functools
import math

import jax
import jax.numpy as jnp
from jax import lax
from jax.experimental import pallas as pl
from jax.experimental.pallas import tpu as pltpu

N_NODES = 10000
N_EDGES = 320000
D_FEAT = 128
D_EDGE_SCALAR = 16
FC_HIDDEN = 64
HALF = D_FEAT // 2

_NB = 2000
_EB = 4000


def _node_linear_body(x_ref, w_ref, nf_ref, self_ref):
    tmp = jnp.dot(x_ref[...], w_ref[...], preferred_element_type=jnp.float32)
    nf = tmp[:, :D_FEAT]
    nf_ref[0] = nf[:, :HALF]
    nf_ref[1] = nf[:, HALF:]
    self_ref[...] = tmp[:, D_FEAT:]


def _node_linear(x, w_scaled):
    grid = (N_NODES // _NB,)
    return pl.pallas_call(
        _node_linear_body,
        grid=grid,
        in_specs=[
            pl.BlockSpec((_NB, D_FEAT), lambda i: (i, 0)),
            pl.BlockSpec((D_FEAT, 2 * D_FEAT), lambda i: (0, 0)),
        ],
        out_specs=[
            pl.BlockSpec((2, _NB, HALF), lambda i: (0, i, 0)),
            pl.BlockSpec((_NB, D_FEAT), lambda i: (i, 0)),
        ],
        out_shape=[
            jax.ShapeDtypeStruct((2, N_NODES, HALF), jnp.float32),
            jax.ShapeDtypeStruct((N_NODES, D_FEAT), jnp.float32),
        ],
    )(x, w_scaled)


def _edge_weights_body(esa_ref, a_ref, wm_ref, wt_ref, out_ref):
    h = jax.nn.gelu(jnp.dot(esa_ref[...], wm_ref[...],
                            preferred_element_type=jnp.float32))
    w = jnp.dot(h, wt_ref[...], preferred_element_type=jnp.float32)
    w = w * a_ref[...]
    out_ref[0] = w[:, :HALF]
    out_ref[1] = w[:, HALF:]


def _edge_weights(esa, a, wm_scaled, wt_scaled):
    grid = (N_EDGES // _EB,)
    return pl.pallas_call(
        _edge_weights_body,
        grid=grid,
        in_specs=[
            pl.BlockSpec((_EB, D_EDGE_SCALAR), lambda i: (i, 0)),
            pl.BlockSpec((_EB, 1), lambda i: (i, 0)),
            pl.BlockSpec((D_EDGE_SCALAR, FC_HIDDEN), lambda i: (0, 0)),
            pl.BlockSpec((FC_HIDDEN, D_FEAT), lambda i: (0, 0)),
        ],
        out_specs=pl.BlockSpec((2, _EB, HALF), lambda i: (0, i, 0)),
        out_shape=jax.ShapeDtypeStruct((2, N_EDGES, HALF), jnp.float32),
    )(esa, a, wm_scaled, wt_scaled)


def _final_mix_body(m0_ref, m1_ref, self_ref, wo_ref, out_ref):
    mid = jnp.concatenate([m0_ref[...], m1_ref[...]], axis=1)
    conv = jnp.dot(mid, wo_ref[...], preferred_element_type=jnp.float32)
    c = math.cos(math.pi / 8.0)
    s = math.sin(math.pi / 8.0)
    out_ref[...] = c * self_ref[...] + s * conv


def _final_mix(m0, m1, self_out, wo_scaled):
    grid = (N_NODES // _NB,)
    return pl.pallas_call(
        _final_mix_body,
        grid=grid,
        in_specs=[
            pl.BlockSpec((_NB, HALF), lambda i: (i, 0)),
            pl.BlockSpec((_NB, HALF), lambda i: (i, 0)),
            pl.BlockSpec((_NB, D_FEAT), lambda i: (i, 0)),
            pl.BlockSpec((D_FEAT, D_FEAT), lambda i: (0, 0)),
        ],
        out_specs=pl.BlockSpec((_NB, D_FEAT), lambda i: (i, 0)),
        out_shape=jax.ShapeDtypeStruct((N_NODES, D_FEAT), jnp.float32),
    )(m0, m1, self_out, wo_scaled)


def kernel(node_input, edge_src, edge_dst, edge_attr, edge_scalar_attr,
           W_self, W_mlp, W_tp, W_out):
    ws = W_self * (1.0 / math.sqrt(D_FEAT))
    wm = W_mlp * (1.0 / math.sqrt(D_EDGE_SCALAR))
    wt = W_tp * (1.0 / math.sqrt(FC_HIDDEN))
    wo = W_out * (1.0 / (math.sqrt(D_FEAT) * math.sqrt(32.0)))

    nf_h, self_out = _node_linear(node_input, ws)
    w_h = _edge_weights(edge_scalar_attr, edge_attr, wm, wt)

    src = edge_src.astype(jnp.int32)
    dst = edge_dst.astype(jnp.int32)
    nf_flat = nf_h.reshape(2 * N_NODES, HALF)
    w_flat = w_h.reshape(2 * N_EDGES, HALF)
    mid = []
    for c in range(2):
        g = jnp.take(nf_flat, src + c * N_NODES, axis=0)
        ef = g * w_flat[c * N_EDGES:(c + 1) * N_EDGES]
        mid.append(jax.ops.segment_sum(ef, dst, num_segments=N_NODES))

    return _final_mix(mid[0], mid[1], self_out, wo)


# speedup vs baseline: 4.2935x; 4.2935x over previous
import math

import jax
import jax.numpy as jnp
from jax import lax
from jax.experimental import pallas as pl
from jax.experimental.pallas import tpu as pltpu
from jax.experimental.pallas import tpu_sc as plsc

N_NODES = 10000
N_EDGES = 320000
D_FEAT = 128
D_EDGE_SCALAR = 16
FC_HIDDEN = 64

_NB = 2000
_EB = 4000


def _node_linear_body(x_ref, w_ref, nf_ref, self_ref):
    tmp = jnp.dot(x_ref[...], w_ref[...], preferred_element_type=jnp.float32)
    nf_ref[...] = tmp[:, :D_FEAT]
    self_ref[...] = tmp[:, D_FEAT:]


def _node_linear(x, w_scaled):
    grid = (N_NODES // _NB,)
    return pl.pallas_call(
        _node_linear_body,
        grid=grid,
        in_specs=[
            pl.BlockSpec((_NB, D_FEAT), lambda i: (i, 0)),
            pl.BlockSpec((D_FEAT, 2 * D_FEAT), lambda i: (0, 0)),
        ],
        out_specs=[
            pl.BlockSpec((_NB, D_FEAT), lambda i: (i, 0)),
            pl.BlockSpec((_NB, D_FEAT), lambda i: (i, 0)),
        ],
        out_shape=[
            jax.ShapeDtypeStruct((N_NODES, D_FEAT), jnp.float32),
            jax.ShapeDtypeStruct((N_NODES, D_FEAT), jnp.float32),
        ],
    )(x, w_scaled)


def _edge_weights_body(esa_ref, a_ref, wm_ref, wt_ref, out_ref):
    h = jax.nn.gelu(jnp.dot(esa_ref[...], wm_ref[...],
                            preferred_element_type=jnp.float32))
    w = jnp.dot(h, wt_ref[...], preferred_element_type=jnp.float32)
    out_ref[...] = w * a_ref[...]


def _edge_weights(esa, a, wm_scaled, wt_scaled):
    grid = (N_EDGES // _EB,)
    return pl.pallas_call(
        _edge_weights_body,
        grid=grid,
        in_specs=[
            pl.BlockSpec((_EB, D_EDGE_SCALAR), lambda i: (i, 0)),
            pl.BlockSpec((_EB, 1), lambda i: (i, 0)),
            pl.BlockSpec((D_EDGE_SCALAR, FC_HIDDEN), lambda i: (0, 0)),
            pl.BlockSpec((FC_HIDDEN, D_FEAT), lambda i: (0, 0)),
        ],
        out_specs=pl.BlockSpec((_EB, D_FEAT), lambda i: (i, 0)),
        out_shape=jax.ShapeDtypeStruct((N_EDGES, D_FEAT), jnp.float32),
    )(esa, a, wm_scaled, wt_scaled)


def _final_mix_body(m0_ref, m1_ref, self_ref, wo_ref, out_ref):
    mid = m0_ref[0] + m1_ref[0]
    conv = jnp.dot(mid, wo_ref[...], preferred_element_type=jnp.float32)
    c = math.cos(math.pi / 8.0)
    s = math.sin(math.pi / 8.0)
    out_ref[...] = c * self_ref[...] + s * conv


def _final_mix(mid2, self_out, wo_scaled):
    grid = (N_NODES // _NB,)
    return pl.pallas_call(
        _final_mix_body,
        grid=grid,
        in_specs=[
            pl.BlockSpec((1, _NB, D_FEAT), lambda i: (0, i, 0)),
            pl.BlockSpec((1, _NB, D_FEAT), lambda i: (1, i, 0)),
            pl.BlockSpec((_NB, D_FEAT), lambda i: (i, 0)),
            pl.BlockSpec((D_FEAT, D_FEAT), lambda i: (0, 0)),
        ],
        out_specs=pl.BlockSpec((_NB, D_FEAT), lambda i: (i, 0)),
        out_shape=jax.ShapeDtypeStruct((N_NODES, D_FEAT), jnp.float32),
    )(mid2, mid2, self_out, wo_scaled)


_CK = 128
_E_PER_CORE = N_EDGES // 2
_NCHUNK = _E_PER_CORE // _CK
_NS = 16
_TILE_STRIDE = 624
_TILE_LEN = 640


def _sc_body(nf_hbm, w_hbm, src_hbm, dst_hbm, out_hbm,
             src_v, dst_v, g_v, w_v, sem, acc_sh):
    c = lax.axis_index("c")
    s = lax.axis_index("s")

    def _zrow(r, carry):
        for j in range(D_FEAT // 16):
            g_v[r, pl.ds(j * 16, 16)] = jnp.zeros((16,), jnp.float32)
        return carry
    lax.fori_loop(0, _CK, _zrow, 0)
    abase = s * _TILE_STRIDE
    for k in range(_TILE_LEN // _CK):
        pltpu.sync_copy(g_v, acc_sh.at[pl.ds(abase + k * _CK, _CK)])
    plsc.subcore_barrier()

    nch = jnp.where(s < _NCHUNK % _NS, _NCHUNK // _NS + 1, _NCHUNK // _NS)
    ebase = c * _E_PER_CORE

    def _chunk(i, carry):
        base = ebase + (s + i * _NS) * _CK
        pltpu.sync_copy(src_hbm.at[pl.ds(base, _CK)], src_v)
        pltpu.sync_copy(dst_hbm.at[pl.ds(base, _CK)], dst_v)

        pltpu.async_copy(nf_hbm.at[src_v], g_v, sem).wait()
        pltpu.sync_copy(w_hbm.at[pl.ds(base, _CK)], w_v)

        def _mrow(r, cy):
            for j in range(D_FEAT // 16):
                sl = pl.ds(j * 16, 16)
                g_v[r, sl] = g_v[r, sl] * w_v[r, sl]
            return cy
        lax.fori_loop(0, _CK, _mrow, 0)

        pltpu.sync_copy(g_v, acc_sh.at[dst_v], add=True)
        return carry
    lax.fori_loop(0, nch, _chunk, 0)

    plsc.subcore_barrier()
    for k in range(_TILE_LEN // _CK):
        pltpu.sync_copy(acc_sh.at[pl.ds(abase + k * _CK, _CK)],
                        out_hbm.at[pl.ds(c * N_NODES + abase + k * _CK, _CK)])


def _sc_gather_scatter(nf, w, src, dst):
    mesh = plsc.VectorSubcoreMesh(core_axis_name="c", subcore_axis_name="s",
                                  num_cores=2, num_subcores=_NS)
    return pl.kernel(
        _sc_body,
        out_type=jax.ShapeDtypeStruct((2 * N_NODES, D_FEAT), jnp.float32),
        mesh=mesh,
        scratch_types=[
            pltpu.VMEM((_CK,), jnp.int32),
            pltpu.VMEM((_CK,), jnp.int32),
            pltpu.VMEM((_CK, D_FEAT), jnp.float32),
            pltpu.VMEM((_CK, D_FEAT), jnp.float32),
            pltpu.SemaphoreType.DMA,
            pltpu.VMEM_SHARED((N_NODES, D_FEAT), jnp.float32),
        ],
    )(nf, w, src, dst)


def kernel(node_input, edge_src, edge_dst, edge_attr, edge_scalar_attr,
           W_self, W_mlp, W_tp, W_out):
    ws = W_self * (1.0 / math.sqrt(D_FEAT))
    wm = W_mlp * (1.0 / math.sqrt(D_EDGE_SCALAR))
    wt = W_tp * (1.0 / math.sqrt(FC_HIDDEN))
    wo = W_out * (1.0 / (math.sqrt(D_FEAT) * math.sqrt(32.0)))

    nf, self_out = _node_linear(node_input, ws)
    w = _edge_weights(edge_scalar_attr, edge_attr, wm, wt)

    src = edge_src.astype(jnp.int32)
    dst = edge_dst.astype(jnp.int32)

    mid = _sc_gather_scatter(nf, w, src, dst)
    mid2 = mid.reshape(2, N_NODES, D_FEAT)

    return _final_mix(mid2, self_out, wo)


# speedup vs baseline: 5.8895x; 1.3717x over previous
import math

import jax
import jax.numpy as jnp
from jax import lax
from jax.experimental import pallas as pl
from jax.experimental.pallas import tpu as pltpu
from jax.experimental.pallas import tpu_sc as plsc

N_NODES = 10000
N_EDGES = 320000
D_FEAT = 128
D_EDGE_SCALAR = 16
FC_HIDDEN = 64

_NB = 2000
_EB = 4000


def _node_linear_body(x_ref, w_ref, nf_ref, self_ref):
    tmp = jnp.dot(x_ref[...], w_ref[...], preferred_element_type=jnp.float32)
    nf_ref[...] = tmp[:, :D_FEAT]
    self_ref[...] = tmp[:, D_FEAT:]


def _node_linear(x, w_scaled):
    grid = (N_NODES // _NB,)
    return pl.pallas_call(
        _node_linear_body,
        grid=grid,
        in_specs=[
            pl.BlockSpec((_NB, D_FEAT), lambda i: (i, 0)),
            pl.BlockSpec((D_FEAT, 2 * D_FEAT), lambda i: (0, 0)),
        ],
        out_specs=[
            pl.BlockSpec((_NB, D_FEAT), lambda i: (i, 0)),
            pl.BlockSpec((_NB, D_FEAT), lambda i: (i, 0)),
        ],
        out_shape=[
            jax.ShapeDtypeStruct((N_NODES, D_FEAT), jnp.float32),
            jax.ShapeDtypeStruct((N_NODES, D_FEAT), jnp.float32),
        ],
    )(x, w_scaled)


def _edge_weights_body(esa_ref, a_ref, wm_ref, wt_ref, out_ref):
    h = jax.nn.gelu(jnp.dot(esa_ref[...], wm_ref[...],
                            preferred_element_type=jnp.float32))
    w = jnp.dot(h, wt_ref[...], preferred_element_type=jnp.float32)
    out_ref[...] = w * a_ref[...]


def _edge_weights(esa, a, wm_scaled, wt_scaled):
    grid = (N_EDGES // _EB,)
    return pl.pallas_call(
        _edge_weights_body,
        grid=grid,
        in_specs=[
            pl.BlockSpec((_EB, D_EDGE_SCALAR), lambda i: (i, 0)),
            pl.BlockSpec((_EB, 1), lambda i: (i, 0)),
            pl.BlockSpec((D_EDGE_SCALAR, FC_HIDDEN), lambda i: (0, 0)),
            pl.BlockSpec((FC_HIDDEN, D_FEAT), lambda i: (0, 0)),
        ],
        out_specs=pl.BlockSpec((_EB, D_FEAT), lambda i: (i, 0)),
        out_shape=jax.ShapeDtypeStruct((N_EDGES, D_FEAT), jnp.float32),
    )(esa, a, wm_scaled, wt_scaled)


def _final_mix_body(m0_ref, m1_ref, self_ref, wo_ref, out_ref):
    mid = m0_ref[0] + m1_ref[0]
    conv = jnp.dot(mid, wo_ref[...], preferred_element_type=jnp.float32)
    c = math.cos(math.pi / 8.0)
    s = math.sin(math.pi / 8.0)
    out_ref[...] = c * self_ref[...] + s * conv


def _final_mix(mid2, self_out, wo_scaled):
    grid = (N_NODES // _NB,)
    return pl.pallas_call(
        _final_mix_body,
        grid=grid,
        in_specs=[
            pl.BlockSpec((1, _NB, D_FEAT), lambda i: (0, i, 0)),
            pl.BlockSpec((1, _NB, D_FEAT), lambda i: (1, i, 0)),
            pl.BlockSpec((_NB, D_FEAT), lambda i: (i, 0)),
            pl.BlockSpec((D_FEAT, D_FEAT), lambda i: (0, 0)),
        ],
        out_specs=pl.BlockSpec((_NB, D_FEAT), lambda i: (i, 0)),
        out_shape=jax.ShapeDtypeStruct((N_NODES, D_FEAT), jnp.float32),
    )(mid2, mid2, self_out, wo_scaled)


_CK = 80
_E_PER_CORE = N_EDGES // 2
_NCHUNK = _E_PER_CORE // _CK
_NS = 16
_TILE_STRIDE = 624
_TILE_LEN = 640


def _sc_body(nf_hbm, w_hbm, src_hbm, dst_hbm, out_hbm,
             src_a, src_b, dst_a, dst_b, g_a, g_b, w_a, w_b,
             isem_a, isem_b, gsem_a, gsem_b, wsem_a, wsem_b, acc_sh):
    c = lax.axis_index("c")
    s = lax.axis_index("s")
    src_v = (src_a, src_b)
    dst_v = (dst_a, dst_b)
    g_v = (g_a, g_b)
    w_v = (w_a, w_b)
    isem = (isem_a, isem_b)
    gsem = (gsem_a, gsem_b)
    wsem = (wsem_a, wsem_b)

    def _zrow(r, carry):
        for j in range(D_FEAT // 16):
            g_a[r, pl.ds(j * 16, 16)] = jnp.zeros((16,), jnp.float32)
        return carry
    lax.fori_loop(0, _CK, _zrow, 0)
    abase = s * _TILE_STRIDE
    for k in range(_TILE_LEN // _CK):
        pltpu.sync_copy(g_a, acc_sh.at[pl.ds(abase + k * _CK, _CK)])
    plsc.subcore_barrier()

    nch = jnp.where(s < _NCHUNK % _NS, _NCHUNK // _NS + 1, _NCHUNK // _NS)
    ebase = c * _E_PER_CORE

    def _ebase_of(i):
        return ebase + (s + i * _NS) * _CK

    def _issue_idx(i, p):
        base = _ebase_of(i)
        pltpu.async_copy(src_hbm.at[pl.ds(base, _CK)], src_v[p], isem[p])
        pltpu.async_copy(dst_hbm.at[pl.ds(base, _CK)], dst_v[p], isem[p])

    def _wait_idx(p):
        pltpu.make_async_copy(src_hbm.at[pl.ds(0, _CK)], src_v[p],
                              isem[p]).wait()
        pltpu.make_async_copy(dst_hbm.at[pl.ds(0, _CK)], dst_v[p],
                              isem[p]).wait()

    def _issue_gw(i, p):
        pltpu.async_copy(nf_hbm.at[src_v[p]], g_v[p], gsem[p])
        pltpu.async_copy(w_hbm.at[pl.ds(_ebase_of(i), _CK)], w_v[p], wsem[p])

    def _wait_gw(p):
        pltpu.make_async_copy(nf_hbm.at[src_v[p]], g_v[p], gsem[p]).wait()
        pltpu.make_async_copy(w_hbm.at[pl.ds(0, _CK)], w_v[p], wsem[p]).wait()

    _issue_idx(0, 0)
    _wait_idx(0)
    _issue_gw(0, 0)
    _issue_idx(1, 1)

    def _body(i, p):
        q = 1 - p

        @pl.when(i < nch)
        def _():
            _wait_gw(p)

            @pl.when(i + 1 < nch)
            def _():
                _wait_idx(q)
                _issue_gw(i + 1, q)

            def _mrow(r, cy):
                for j in range(D_FEAT // 16):
                    sl = pl.ds(j * 16, 16)
                    g_v[p][r, sl] = g_v[p][r, sl] * w_v[p][r, sl]
                return cy
            lax.fori_loop(0, _CK, _mrow, 0)

            pltpu.sync_copy(g_v[p], acc_sh.at[dst_v[p]], add=True)

            @pl.when(i + 2 < nch)
            def _():
                _issue_idx(i + 2, p)

    def _pair(k, carry):
        _body(2 * k, 0)
        _body(2 * k + 1, 1)
        return carry
    lax.fori_loop(0, (nch + 1) // 2, _pair, 0)

    plsc.subcore_barrier()
    for k in range(_TILE_LEN // _CK):
        pltpu.sync_copy(acc_sh.at[pl.ds(abase + k * _CK, _CK)],
                        out_hbm.at[pl.ds(c * N_NODES + abase + k * _CK, _CK)])


def _sc_gather_scatter(nf, w, src, dst):
    mesh = plsc.VectorSubcoreMesh(core_axis_name="c", subcore_axis_name="s",
                                  num_cores=2, num_subcores=_NS)
    return pl.kernel(
        _sc_body,
        out_type=jax.ShapeDtypeStruct((2 * N_NODES, D_FEAT), jnp.float32),
        mesh=mesh,
        scratch_types=[
            pltpu.VMEM((_CK,), jnp.int32),
            pltpu.VMEM((_CK,), jnp.int32),
            pltpu.VMEM((_CK,), jnp.int32),
            pltpu.VMEM((_CK,), jnp.int32),
            pltpu.VMEM((_CK, D_FEAT), jnp.float32),
            pltpu.VMEM((_CK, D_FEAT), jnp.float32),
            pltpu.VMEM((_CK, D_FEAT), jnp.float32),
            pltpu.VMEM((_CK, D_FEAT), jnp.float32),
            pltpu.SemaphoreType.DMA,
            pltpu.SemaphoreType.DMA,
            pltpu.SemaphoreType.DMA,
            pltpu.SemaphoreType.DMA,
            pltpu.SemaphoreType.DMA,
            pltpu.SemaphoreType.DMA,
            pltpu.VMEM_SHARED((N_NODES, D_FEAT), jnp.float32),
        ],
    )(nf, w, src, dst)


def kernel(node_input, edge_src, edge_dst, edge_attr, edge_scalar_attr,
           W_self, W_mlp, W_tp, W_out):
    ws = W_self * (1.0 / math.sqrt(D_FEAT))
    wm = W_mlp * (1.0 / math.sqrt(D_EDGE_SCALAR))
    wt = W_tp * (1.0 / math.sqrt(FC_HIDDEN))
    wo = W_out * (1.0 / (math.sqrt(D_FEAT) * math.sqrt(32.0)))

    nf, self_out = _node_linear(node_input, ws)
    w = _edge_weights(edge_scalar_attr, edge_attr, wm, wt)

    src = edge_src.astype(jnp.int32)
    dst = edge_dst.astype(jnp.int32)

    mid = _sc_gather_scatter(nf, w, src, dst)
    mid2 = mid.reshape(2, N_NODES, D_FEAT)

    return _final_mix(mid2, self_out, wo)


# speedup vs baseline: 8.7587x; 1.4872x over previous
import math

import jax
import jax.numpy as jnp
from jax import lax
from jax.experimental import pallas as pl
from jax.experimental.pallas import tpu as pltpu
from jax.experimental.pallas import tpu_sc as plsc

N_NODES = 10000
N_EDGES = 320000
D_FEAT = 128
D_EDGE_SCALAR = 16
FC_HIDDEN = 64

_NB = 2000
_EB = 4096


def _node_linear_body(x_ref, w_ref, nf_ref, self_ref):
    tmp = jnp.dot(x_ref[...], w_ref[...], preferred_element_type=jnp.float32)
    nf_ref[...] = tmp[:, :D_FEAT]
    self_ref[...] = tmp[:, D_FEAT:]


def _node_linear(x, w_scaled):
    grid = (N_NODES // _NB,)
    return pl.pallas_call(
        _node_linear_body,
        grid=grid,
        in_specs=[
            pl.BlockSpec((_NB, D_FEAT), lambda i: (i, 0)),
            pl.BlockSpec((D_FEAT, 2 * D_FEAT), lambda i: (0, 0)),
        ],
        out_specs=[
            pl.BlockSpec((_NB, D_FEAT), lambda i: (i, 0)),
            pl.BlockSpec((_NB, D_FEAT), lambda i: (i, 0)),
        ],
        out_shape=[
            jax.ShapeDtypeStruct((N_NODES, D_FEAT), jnp.float32),
            jax.ShapeDtypeStruct((N_NODES, D_FEAT), jnp.float32),
        ],
    )(x, w_scaled)


def _edge_weights_body(esat_ref, wmt_ref, wt_ref, out_ref):
    ht = jax.nn.gelu(jnp.dot(wmt_ref[...], esat_ref[...],
                             preferred_element_type=jnp.float32))
    out_ref[...] = jnp.dot(ht.T, wt_ref[...],
                           preferred_element_type=jnp.float32)


def _edge_weights(esat, wmt_scaled, wt_scaled):
    grid = (pl.cdiv(N_EDGES, _EB),)
    return pl.pallas_call(
        _edge_weights_body,
        grid=grid,
        in_specs=[
            pl.BlockSpec((D_EDGE_SCALAR, _EB), lambda i: (0, i)),
            pl.BlockSpec((FC_HIDDEN, D_EDGE_SCALAR), lambda i: (0, 0)),
            pl.BlockSpec((FC_HIDDEN, D_FEAT), lambda i: (0, 0)),
        ],
        out_specs=pl.BlockSpec((_EB, D_FEAT), lambda i: (i, 0)),
        out_shape=jax.ShapeDtypeStruct((N_EDGES, D_FEAT), jnp.float32),
    )(esat, wmt_scaled, wt_scaled)


def _final_mix_body(m0_ref, m1_ref, self_ref, wo_ref, out_ref):
    mid = m0_ref[0] + m1_ref[0]
    conv = jnp.dot(mid, wo_ref[...], preferred_element_type=jnp.float32)
    c = math.cos(math.pi / 8.0)
    s = math.sin(math.pi / 8.0)
    out_ref[...] = c * self_ref[...] + s * conv


def _final_mix(mid2, self_out, wo_scaled):
    grid = (N_NODES // _NB,)
    return pl.pallas_call(
        _final_mix_body,
        grid=grid,
        in_specs=[
            pl.BlockSpec((1, _NB, D_FEAT), lambda i: (0, i, 0)),
            pl.BlockSpec((1, _NB, D_FEAT), lambda i: (1, i, 0)),
            pl.BlockSpec((_NB, D_FEAT), lambda i: (i, 0)),
            pl.BlockSpec((D_FEAT, D_FEAT), lambda i: (0, 0)),
        ],
        out_specs=pl.BlockSpec((_NB, D_FEAT), lambda i: (i, 0)),
        out_shape=jax.ShapeDtypeStruct((N_NODES, D_FEAT), jnp.float32),
    )(mid2, mid2, self_out, wo_scaled)


_CK = 80
_E_PER_CORE = N_EDGES // 2
_NCHUNK = _E_PER_CORE // _CK
_NS = 16
_TILE_STRIDE = 624
_TILE_LEN = 640


def _sc_body(nf_hbm, w_hbm, a_hbm, src_hbm, dst_hbm, out_hbm,
             src_a, src_b, dst_a, dst_b, g_a, g_b, w_a, w_b, a_a, a_b,
             isem_a, isem_b, gsem_a, gsem_b, wsem_a, wsem_b, acc_sh):
    c = lax.axis_index("c")
    s = lax.axis_index("s")
    src_v = (src_a, src_b)
    dst_v = (dst_a, dst_b)
    g_v = (g_a, g_b)
    w_v = (w_a, w_b)
    a_v = (a_a, a_b)
    isem = (isem_a, isem_b)
    gsem = (gsem_a, gsem_b)
    wsem = (wsem_a, wsem_b)

    def _zrow(r, carry):
        for j in range(D_FEAT // 16):
            g_a[r, pl.ds(j * 16, 16)] = jnp.zeros((16,), jnp.float32)
        return carry
    lax.fori_loop(0, _CK, _zrow, 0)
    abase = s * _TILE_STRIDE
    for k in range(_TILE_LEN // _CK):
        pltpu.sync_copy(g_a, acc_sh.at[pl.ds(abase + k * _CK, _CK)])
    plsc.subcore_barrier()

    nch = jnp.where(s < _NCHUNK % _NS, _NCHUNK // _NS + 1, _NCHUNK // _NS)
    ebase = c * _E_PER_CORE

    def _ebase_of(i):
        return ebase + (s + i * _NS) * _CK

    def _issue_idx(i, p):
        base = _ebase_of(i)
        pltpu.async_copy(src_hbm.at[pl.ds(base, _CK)], src_v[p], isem[p])
        pltpu.async_copy(dst_hbm.at[pl.ds(base, _CK)], dst_v[p], isem[p])
        pltpu.async_copy(a_hbm.at[pl.ds(base, _CK)], a_v[p], isem[p])

    def _wait_idx(p):
        pltpu.make_async_copy(src_hbm.at[pl.ds(0, _CK)], src_v[p],
                              isem[p]).wait()
        pltpu.make_async_copy(dst_hbm.at[pl.ds(0, _CK)], dst_v[p],
                              isem[p]).wait()
        pltpu.make_async_copy(a_hbm.at[pl.ds(0, _CK)], a_v[p],
                              isem[p]).wait()

    def _issue_gw(i, p):
        pltpu.async_copy(nf_hbm.at[src_v[p]], g_v[p], gsem[p])
        pltpu.async_copy(w_hbm.at[pl.ds(_ebase_of(i), _CK)], w_v[p], wsem[p])

    def _wait_gw(p):
        pltpu.make_async_copy(nf_hbm.at[src_v[p]], g_v[p], gsem[p]).wait()
        pltpu.make_async_copy(w_hbm.at[pl.ds(0, _CK)], w_v[p], wsem[p]).wait()

    _issue_idx(0, 0)
    _wait_idx(0)
    _issue_gw(0, 0)
    _issue_idx(1, 1)

    def _body(i, p):
        q = 1 - p

        @pl.when(i < nch)
        def _():
            _wait_gw(p)

            @pl.when(i + 1 < nch)
            def _():
                _wait_idx(q)
                _issue_gw(i + 1, q)

            def _mrow(r, cy):
                ar = a_v[p][pl.ds(r, 1)][0]
                for j in range(D_FEAT // 16):
                    sl = pl.ds(j * 16, 16)
                    g_v[p][r, sl] = g_v[p][r, sl] * w_v[p][r, sl] * ar
                return cy
            lax.fori_loop(0, _CK, _mrow, 0)

            pltpu.sync_copy(g_v[p], acc_sh.at[dst_v[p]], add=True)

            @pl.when(i + 2 < nch)
            def _():
                _issue_idx(i + 2, p)

    def _pair(k, carry):
        _body(2 * k, 0)
        _body(2 * k + 1, 1)
        return carry
    lax.fori_loop(0, (nch + 1) // 2, _pair, 0)

    plsc.subcore_barrier()
    for k in range(_TILE_LEN // _CK):
        pltpu.sync_copy(acc_sh.at[pl.ds(abase + k * _CK, _CK)],
                        out_hbm.at[pl.ds(c * N_NODES + abase + k * _CK, _CK)])


def _sc_gather_scatter(nf, w, a, src, dst):
    mesh = plsc.VectorSubcoreMesh(core_axis_name="c", subcore_axis_name="s",
                                  num_cores=2, num_subcores=_NS)
    return pl.kernel(
        _sc_body,
        out_type=jax.ShapeDtypeStruct((2 * N_NODES, D_FEAT), jnp.float32),
        mesh=mesh,
        scratch_types=[
            pltpu.VMEM((_CK,), jnp.int32),
            pltpu.VMEM((_CK,), jnp.int32),
            pltpu.VMEM((_CK,), jnp.int32),
            pltpu.VMEM((_CK,), jnp.int32),
            pltpu.VMEM((_CK, D_FEAT), jnp.float32),
            pltpu.VMEM((_CK, D_FEAT), jnp.float32),
            pltpu.VMEM((_CK, D_FEAT), jnp.float32),
            pltpu.VMEM((_CK, D_FEAT), jnp.float32),
            pltpu.VMEM((_CK,), jnp.float32),
            pltpu.VMEM((_CK,), jnp.float32),
            pltpu.SemaphoreType.DMA,
            pltpu.SemaphoreType.DMA,
            pltpu.SemaphoreType.DMA,
            pltpu.SemaphoreType.DMA,
            pltpu.SemaphoreType.DMA,
            pltpu.SemaphoreType.DMA,
            pltpu.VMEM_SHARED((N_NODES, D_FEAT), jnp.float32),
        ],
    )(nf, w, a, src, dst)


def kernel(node_input, edge_src, edge_dst, edge_attr, edge_scalar_attr,
           W_self, W_mlp, W_tp, W_out):
    ws = W_self * (1.0 / math.sqrt(D_FEAT))
    wm = W_mlp * (1.0 / math.sqrt(D_EDGE_SCALAR))
    wt = W_tp * (1.0 / math.sqrt(FC_HIDDEN))
    wo = W_out * (1.0 / (math.sqrt(D_FEAT) * math.sqrt(32.0)))

    nf, self_out = _node_linear(node_input, ws)
    esat = edge_scalar_attr.T
    w = _edge_weights(esat, wm.T, wt)

    src = edge_src.astype(jnp.int32)
    dst = edge_dst.astype(jnp.int32)
    a_flat = edge_attr.reshape(N_EDGES)

    mid = _sc_gather_scatter(nf, w, a_flat, src, dst)
    mid2 = mid.reshape(2, N_NODES, D_FEAT)

    return _final_mix(mid2, self_out, wo)


# speedup vs baseline: 9.4433x; 1.0782x over previous
import math

import jax
import jax.numpy as jnp
from jax import lax
from jax.experimental import pallas as pl
from jax.experimental.pallas import tpu as pltpu
from jax.experimental.pallas import tpu_sc as plsc

N_NODES = 10000
N_EDGES = 320000
D_FEAT = 128
D_EDGE_SCALAR = 16
FC_HIDDEN = 64

_NB = 2000
_EB = 4096


def _node_linear_body(x_ref, w_ref, nf_ref, self_ref):
    tmp = jnp.dot(x_ref[...], w_ref[...], preferred_element_type=jnp.float32)
    nf_ref[...] = tmp[:, :D_FEAT]
    self_ref[...] = tmp[:, D_FEAT:]


def _node_linear(x, w_scaled):
    grid = (N_NODES // _NB,)
    return pl.pallas_call(
        _node_linear_body,
        grid=grid,
        in_specs=[
            pl.BlockSpec((_NB, D_FEAT), lambda i: (i, 0)),
            pl.BlockSpec((D_FEAT, 2 * D_FEAT), lambda i: (0, 0)),
        ],
        out_specs=[
            pl.BlockSpec((_NB, D_FEAT), lambda i: (i, 0)),
            pl.BlockSpec((_NB, D_FEAT), lambda i: (i, 0)),
        ],
        out_shape=[
            jax.ShapeDtypeStruct((N_NODES, D_FEAT), jnp.float32),
            jax.ShapeDtypeStruct((N_NODES, D_FEAT), jnp.float32),
        ],
    )(x, w_scaled)


def _edge_weights_body(esat_ref, wmt_ref, wt_ref, out_ref):
    ht = jax.nn.gelu(jnp.dot(wmt_ref[...], esat_ref[...],
                             preferred_element_type=jnp.float32))
    out_ref[...] = jnp.dot(ht.T, wt_ref[...],
                           preferred_element_type=jnp.float32)


def _edge_weights(esat, wmt_scaled, wt_scaled):
    grid = (pl.cdiv(N_EDGES, _EB),)
    return pl.pallas_call(
        _edge_weights_body,
        grid=grid,
        in_specs=[
            pl.BlockSpec((D_EDGE_SCALAR, _EB), lambda i: (0, i)),
            pl.BlockSpec((FC_HIDDEN, D_EDGE_SCALAR), lambda i: (0, 0)),
            pl.BlockSpec((FC_HIDDEN, D_FEAT), lambda i: (0, 0)),
        ],
        out_specs=pl.BlockSpec((_EB, D_FEAT), lambda i: (i, 0)),
        out_shape=jax.ShapeDtypeStruct((N_EDGES, D_FEAT), jnp.float32),
    )(esat, wmt_scaled, wt_scaled)


def _final_mix_body(m0_ref, m1_ref, self_ref, wo_ref, out_ref):
    mid = m0_ref[0] + m1_ref[0]
    conv = jnp.dot(mid, wo_ref[...], preferred_element_type=jnp.float32)
    c = math.cos(math.pi / 8.0)
    s = math.sin(math.pi / 8.0)
    out_ref[...] = c * self_ref[...] + s * conv


def _final_mix(mid2, self_out, wo_scaled):
    grid = (N_NODES // _NB,)
    return pl.pallas_call(
        _final_mix_body,
        grid=grid,
        in_specs=[
            pl.BlockSpec((1, _NB, D_FEAT), lambda i: (0, i, 0)),
            pl.BlockSpec((1, _NB, D_FEAT), lambda i: (1, i, 0)),
            pl.BlockSpec((_NB, D_FEAT), lambda i: (i, 0)),
            pl.BlockSpec((D_FEAT, D_FEAT), lambda i: (0, 0)),
        ],
        out_specs=pl.BlockSpec((_NB, D_FEAT), lambda i: (i, 0)),
        out_shape=jax.ShapeDtypeStruct((N_NODES, D_FEAT), jnp.float32),
    )(mid2, mid2, self_out, wo_scaled)


_CK = 80
_E_PER_CORE = N_EDGES // 2
_NCH = _E_PER_CORE // _CK // 16
_NS = 16
_TILE_STRIDE = 624
_TILE_LEN = 640


def _sc_body(nf_hbm, w_hbm, a_hbm, src_hbm, dst_hbm, out_hbm,
             src_0, src_1, src_2, dst_0, dst_1, dst_2, a_0, a_1, a_2,
             g_a, g_b, w_a, w_b,
             isem_0, isem_1, isem_2, gsem_a, gsem_b, wsem_a, wsem_b,
             ssem_a, ssem_b, acc_sh):
    c = lax.axis_index("c")
    s = lax.axis_index("s")
    src_v = (src_0, src_1, src_2)
    dst_v = (dst_0, dst_1, dst_2)
    a_v = (a_0, a_1, a_2)
    g_v = (g_a, g_b)
    w_v = (w_a, w_b)
    isem = (isem_0, isem_1, isem_2)
    gsem = (gsem_a, gsem_b)
    wsem = (wsem_a, wsem_b)
    ssem = (ssem_a, ssem_b)

    def _zrow(r, carry):
        for j in range(D_FEAT // 16):
            g_a[r, pl.ds(j * 16, 16)] = jnp.zeros((16,), jnp.float32)
        return carry
    lax.fori_loop(0, _CK, _zrow, 0)
    abase = s * _TILE_STRIDE
    for k in range(_TILE_LEN // _CK):
        pltpu.sync_copy(g_a, acc_sh.at[pl.ds(abase + k * _CK, _CK)])
    plsc.subcore_barrier()

    ebase = c * _E_PER_CORE

    def _chunk_base(i):
        return ebase + (s + i * _NS) * _CK

    def _issue_idx(i, t):
        base = _chunk_base(i)
        pltpu.async_copy(src_hbm.at[pl.ds(base, _CK)], src_v[t], isem[t])
        pltpu.async_copy(dst_hbm.at[pl.ds(base, _CK)], dst_v[t], isem[t])
        pltpu.async_copy(a_hbm.at[pl.ds(base, _CK)], a_v[t], isem[t])

    def _wait_idx(t):
        pltpu.make_async_copy(src_hbm.at[pl.ds(0, _CK)], src_v[t],
                              isem[t]).wait()
        pltpu.make_async_copy(dst_hbm.at[pl.ds(0, _CK)], dst_v[t],
                              isem[t]).wait()
        pltpu.make_async_copy(a_hbm.at[pl.ds(0, _CK)], a_v[t],
                              isem[t]).wait()

    def _issue_gw(i, t, p):
        pltpu.async_copy(nf_hbm.at[src_v[t]], g_v[p], gsem[p])
        pltpu.async_copy(w_hbm.at[pl.ds(_chunk_base(i), _CK)], w_v[p],
                         wsem[p])

    def _wait_gw(t, p):
        pltpu.make_async_copy(nf_hbm.at[src_v[t]], g_v[p], gsem[p]).wait()
        pltpu.make_async_copy(w_hbm.at[pl.ds(0, _CK)], w_v[p],
                              wsem[p]).wait()

    def _mul(t, p):
        def _mrow(r4, cy):
            for rr in range(4):
                r = r4 * 4 + rr
                ar = a_v[t][pl.ds(r, 1)][0]
                for j in range(D_FEAT // 16):
                    sl = pl.ds(j * 16, 16)
                    g_v[p][r, sl] = g_v[p][r, sl] * w_v[p][r, sl] * ar
            return cy
        lax.fori_loop(0, _CK // 4, _mrow, 0)

    def _scatter(t, p):
        pltpu.async_copy(g_v[p], acc_sh.at[dst_v[t]], ssem[p], add=True)

    def _wait_scatter(p):
        pltpu.make_async_copy(w_hbm.at[pl.ds(0, _CK)], g_v[p],
                              ssem[p]).wait()

    _issue_idx(0, 0)
    _wait_idx(0)
    _issue_gw(0, 0, 0)
    _issue_idx(1, 1)

    _wait_gw(0, 0)
    _wait_idx(1)
    _issue_gw(1, 1, 1)
    _issue_idx(2, 2)
    _mul(0, 0)
    _scatter(0, 0)

    def _steady(i, t, tn, tn2, p, q):
        _wait_gw(t, p)
        _wait_idx(tn)
        _wait_scatter(q)
        _issue_gw(i + 1, tn, q)
        _issue_idx(i + 2, tn2)
        _mul(t, p)
        _scatter(t, p)

    def _six(k, carry):
        for o in range(6):
            i0 = 1 + o
            _steady(6 * k + i0, i0 % 3, (i0 + 1) % 3, (i0 + 2) % 3,
                    i0 % 2, (i0 + 1) % 2)
        return carry
    n_six = (_NCH - 2) // 6
    lax.fori_loop(0, n_six, _six, 0)
    for i0 in range(1 + 6 * n_six, _NCH - 1):
        _steady(i0, i0 % 3, (i0 + 1) % 3, (i0 + 2) % 3, i0 % 2, (i0 + 1) % 2)

    iL = _NCH - 1
    _wait_gw(iL % 3, iL % 2)
    _mul(iL % 3, iL % 2)
    _scatter(iL % 3, iL % 2)
    _wait_scatter(0)
    _wait_scatter(1)

    plsc.subcore_barrier()
    for k in range(_TILE_LEN // _CK):
        pltpu.sync_copy(acc_sh.at[pl.ds(abase + k * _CK, _CK)],
                        out_hbm.at[pl.ds(c * N_NODES + abase + k * _CK, _CK)])


def _sc_gather_scatter(nf, w, a, src, dst):
    mesh = plsc.VectorSubcoreMesh(core_axis_name="c", subcore_axis_name="s",
                                  num_cores=2, num_subcores=_NS)
    return pl.kernel(
        _sc_body,
        out_type=jax.ShapeDtypeStruct((2 * N_NODES, D_FEAT), jnp.float32),
        mesh=mesh,
        scratch_types=(
            [pltpu.VMEM((_CK,), jnp.int32)] * 6
            + [pltpu.VMEM((_CK,), jnp.float32)] * 3
            + [pltpu.VMEM((_CK, D_FEAT), jnp.float32)] * 4
            + [pltpu.SemaphoreType.DMA] * 9
            + [pltpu.VMEM_SHARED((N_NODES, D_FEAT), jnp.float32)]
        ),
    )(nf, w, a, src, dst)


def kernel(node_input, edge_src, edge_dst, edge_attr, edge_scalar_attr,
           W_self, W_mlp, W_tp, W_out):
    ws = W_self * (1.0 / math.sqrt(D_FEAT))
    wm = W_mlp * (1.0 / math.sqrt(D_EDGE_SCALAR))
    wt = W_tp * (1.0 / math.sqrt(FC_HIDDEN))
    wo = W_out * (1.0 / (math.sqrt(D_FEAT) * math.sqrt(32.0)))

    nf, self_out = _node_linear(node_input, ws)
    esat = edge_scalar_attr.T
    w = _edge_weights(esat, wm.T, wt)

    src = edge_src.astype(jnp.int32)
    dst = edge_dst.astype(jnp.int32)
    a_flat = edge_attr.reshape(N_EDGES)

    mid = _sc_gather_scatter(nf, w, a_flat, src, dst)
    mid2 = mid.reshape(2, N_NODES, D_FEAT)

    return _final_mix(mid2, self_out, wo)


# speedup vs baseline: 9.8777x; 1.0460x over previous
import math

import jax
import jax.numpy as jnp
from jax import lax
from jax.experimental import pallas as pl
from jax.experimental.pallas import tpu as pltpu
from jax.experimental.pallas import tpu_sc as plsc

N_NODES = 10000
N_EDGES = 320000
D_FEAT = 128
D_EDGE_SCALAR = 16
FC_HIDDEN = 64

_NB = 2000
_EB = 4096


def _node_linear_body(x_ref, w_ref, nf_ref, self_ref):
    tmp = jnp.dot(x_ref[...], w_ref[...], preferred_element_type=jnp.float32)
    nf_ref[...] = tmp[:, :D_FEAT]
    self_ref[...] = tmp[:, D_FEAT:]


def _node_linear(x, w_scaled):
    grid = (N_NODES // _NB,)
    return pl.pallas_call(
        _node_linear_body,
        grid=grid,
        in_specs=[
            pl.BlockSpec((_NB, D_FEAT), lambda i: (i, 0)),
            pl.BlockSpec((D_FEAT, 2 * D_FEAT), lambda i: (0, 0)),
        ],
        out_specs=[
            pl.BlockSpec((_NB, D_FEAT), lambda i: (i, 0)),
            pl.BlockSpec((_NB, D_FEAT), lambda i: (i, 0)),
        ],
        out_shape=[
            jax.ShapeDtypeStruct((N_NODES, D_FEAT), jnp.float32),
            jax.ShapeDtypeStruct((N_NODES, D_FEAT), jnp.float32),
        ],
    )(x, w_scaled)


def _edge_weights_body(esat_ref, at_ref, wmt_ref, wt_ref, out_ref):
    ht = jax.nn.gelu(jnp.dot(wmt_ref[...], esat_ref[...],
                             preferred_element_type=jnp.float32))
    ht = ht * at_ref[...]
    out_ref[...] = jnp.dot(ht.T, wt_ref[...],
                           preferred_element_type=jnp.float32)


def _edge_weights(esat, at, wmt_scaled, wt_scaled):
    grid = (pl.cdiv(N_EDGES, _EB),)
    return pl.pallas_call(
        _edge_weights_body,
        grid=grid,
        in_specs=[
            pl.BlockSpec((D_EDGE_SCALAR, _EB), lambda i: (0, i)),
            pl.BlockSpec((1, _EB), lambda i: (0, i)),
            pl.BlockSpec((FC_HIDDEN, D_EDGE_SCALAR), lambda i: (0, 0)),
            pl.BlockSpec((FC_HIDDEN, D_FEAT), lambda i: (0, 0)),
        ],
        out_specs=pl.BlockSpec((_EB, D_FEAT), lambda i: (i, 0)),
        out_shape=jax.ShapeDtypeStruct((N_EDGES, D_FEAT), jnp.float32),
    )(esat, at, wmt_scaled, wt_scaled)


def _final_mix_body(m0_ref, m1_ref, self_ref, wo_ref, out_ref):
    mid = m0_ref[0] + m1_ref[0]
    conv = jnp.dot(mid, wo_ref[...], preferred_element_type=jnp.float32)
    c = math.cos(math.pi / 8.0)
    s = math.sin(math.pi / 8.0)
    out_ref[...] = c * self_ref[...] + s * conv


def _final_mix(mid2, self_out, wo_scaled):
    grid = (N_NODES // _NB,)
    return pl.pallas_call(
        _final_mix_body,
        grid=grid,
        in_specs=[
            pl.BlockSpec((1, _NB, D_FEAT), lambda i: (0, i, 0)),
            pl.BlockSpec((1, _NB, D_FEAT), lambda i: (1, i, 0)),
            pl.BlockSpec((_NB, D_FEAT), lambda i: (i, 0)),
            pl.BlockSpec((D_FEAT, D_FEAT), lambda i: (0, 0)),
        ],
        out_specs=pl.BlockSpec((_NB, D_FEAT), lambda i: (i, 0)),
        out_shape=jax.ShapeDtypeStruct((N_NODES, D_FEAT), jnp.float32),
    )(mid2, mid2, self_out, wo_scaled)


_CK = 80
_E_PER_CORE = N_EDGES // 2
_NCH = _E_PER_CORE // _CK // 16
_NS = 16
_TILE_STRIDE = 624
_TILE_LEN = 640


def _sc_body(nf_hbm, w_hbm, src_hbm, dst_hbm, out_hbm,
             src_0, src_1, src_2, dst_0, dst_1, dst_2,
             g_a, g_b, w_a, w_b,
             isem_0, isem_1, isem_2, gsem_a, gsem_b, wsem_a, wsem_b,
             ssem_a, ssem_b, acc_sh):
    c = lax.axis_index("c")
    s = lax.axis_index("s")
    src_v = (src_0, src_1, src_2)
    dst_v = (dst_0, dst_1, dst_2)
    g_v = (g_a, g_b)
    w_v = (w_a, w_b)
    isem = (isem_0, isem_1, isem_2)
    gsem = (gsem_a, gsem_b)
    wsem = (wsem_a, wsem_b)
    ssem = (ssem_a, ssem_b)

    def _zrow(r, carry):
        for j in range(D_FEAT // 16):
            g_a[r, pl.ds(j * 16, 16)] = jnp.zeros((16,), jnp.float32)
        return carry
    lax.fori_loop(0, _CK, _zrow, 0)
    abase = s * _TILE_STRIDE
    for k in range(_TILE_LEN // _CK):
        pltpu.sync_copy(g_a, acc_sh.at[pl.ds(abase + k * _CK, _CK)])
    plsc.subcore_barrier()

    ebase = c * _E_PER_CORE

    def _chunk_base(i):
        return ebase + (s + i * _NS) * _CK

    def _issue_idx(i, t):
        base = _chunk_base(i)
        pltpu.async_copy(src_hbm.at[pl.ds(base, _CK)], src_v[t], isem[t])
        pltpu.async_copy(dst_hbm.at[pl.ds(base, _CK)], dst_v[t], isem[t])

    def _wait_idx(t):
        pltpu.make_async_copy(src_hbm.at[pl.ds(0, _CK)], src_v[t],
                              isem[t]).wait()
        pltpu.make_async_copy(dst_hbm.at[pl.ds(0, _CK)], dst_v[t],
                              isem[t]).wait()

    def _issue_gw(i, t, p):
        pltpu.async_copy(nf_hbm.at[src_v[t]], g_v[p], gsem[p])
        pltpu.async_copy(w_hbm.at[pl.ds(_chunk_base(i), _CK)], w_v[p],
                         wsem[p])

    def _wait_gw(t, p):
        pltpu.make_async_copy(nf_hbm.at[src_v[t]], g_v[p], gsem[p]).wait()
        pltpu.make_async_copy(w_hbm.at[pl.ds(0, _CK)], w_v[p],
                              wsem[p]).wait()

    def _mul(t, p):
        def _mrow(r4, cy):
            for rr in range(4):
                r = r4 * 4 + rr
                for j in range(D_FEAT // 16):
                    sl = pl.ds(j * 16, 16)
                    g_v[p][r, sl] = g_v[p][r, sl] * w_v[p][r, sl]
            return cy
        lax.fori_loop(0, _CK // 4, _mrow, 0)

    def _scatter(t, p):
        pltpu.async_copy(g_v[p], acc_sh.at[dst_v[t]], ssem[p], add=True)

    def _wait_scatter(p):
        pltpu.make_async_copy(nf_hbm.at[pl.ds(0, _CK)], g_v[p],
                              ssem[p]).wait()

    _issue_idx(0, 0)
    _wait_idx(0)
    _issue_gw(0, 0, 0)
    _issue_idx(1, 1)

    _wait_gw(0, 0)
    _wait_idx(1)
    _issue_gw(1, 1, 1)
    _issue_idx(2, 2)
    _mul(0, 0)
    _scatter(0, 0)

    def _steady(i, t, tn, tn2, p, q):
        _wait_gw(t, p)
        _wait_idx(tn)
        _wait_scatter(q)
        _issue_gw(i + 1, tn, q)
        _issue_idx(i + 2, tn2)
        _mul(t, p)
        _scatter(t, p)

    def _six(k, carry):
        for o in range(6):
            i0 = 1 + o
            _steady(6 * k + i0, i0 % 3, (i0 + 1) % 3, (i0 + 2) % 3,
                    i0 % 2, (i0 + 1) % 2)
        return carry
    n_six = (_NCH - 2) // 6
    lax.fori_loop(0, n_six, _six, 0)
    for i0 in range(1 + 6 * n_six, _NCH - 1):
        _steady(i0, i0 % 3, (i0 + 1) % 3, (i0 + 2) % 3, i0 % 2, (i0 + 1) % 2)

    iL = _NCH - 1
    _wait_gw(iL % 3, iL % 2)
    _mul(iL % 3, iL % 2)
    _scatter(iL % 3, iL % 2)
    _wait_scatter(0)
    _wait_scatter(1)

    plsc.subcore_barrier()
    for k in range(_TILE_LEN // _CK):
        pltpu.sync_copy(acc_sh.at[pl.ds(abase + k * _CK, _CK)],
                        out_hbm.at[pl.ds(c * N_NODES + abase + k * _CK, _CK)])


def _sc_gather_scatter(nf, w, src, dst):
    mesh = plsc.VectorSubcoreMesh(core_axis_name="c", subcore_axis_name="s",
                                  num_cores=2, num_subcores=_NS)
    return pl.kernel(
        _sc_body,
        out_type=jax.ShapeDtypeStruct((2 * N_NODES, D_FEAT), jnp.float32),
        mesh=mesh,
        scratch_types=(
            [pltpu.VMEM((_CK,), jnp.int32)] * 6
            + [pltpu.VMEM((_CK, D_FEAT), jnp.float32)] * 4
            + [pltpu.SemaphoreType.DMA] * 9
            + [pltpu.VMEM_SHARED((N_NODES, D_FEAT), jnp.float32)]
        ),
    )(nf, w, src, dst)


def kernel(node_input, edge_src, edge_dst, edge_attr, edge_scalar_attr,
           W_self, W_mlp, W_tp, W_out):
    ws = W_self * (1.0 / math.sqrt(D_FEAT))
    wm = W_mlp * (1.0 / math.sqrt(D_EDGE_SCALAR))
    wt = W_tp * (1.0 / math.sqrt(FC_HIDDEN))
    wo = W_out * (1.0 / (math.sqrt(D_FEAT) * math.sqrt(32.0)))

    nf, self_out = _node_linear(node_input, ws)
    esat = edge_scalar_attr.T
    at = edge_attr.reshape(1, N_EDGES)
    w = _edge_weights(esat, at, wm.T, wt)

    src = edge_src.astype(jnp.int32)
    dst = edge_dst.astype(jnp.int32)

    mid = _sc_gather_scatter(nf, w, src, dst)
    mid2 = mid.reshape(2, N_NODES, D_FEAT)

    return _final_mix(mid2, self_out, wo)
